# baseline (speedup 1.0000x reference)
import jax
import jax.numpy as jnp
from jax import lax
from jax.experimental import pallas as pl
from jax.experimental.pallas import tpu as pltpu, tpu_sc as plsc

B, F, V, D = 16384, 26, 1000000, 16
FP = 32
SDP = FP * D
H1, H2, H3 = 1024, 512, 256
EPS = 1e-5

_NC, _NS = 2, 16
_NW = _NC * _NS
_N = B * FP
_PER_W = _N // _NW
_NCHUNK = 8
_CH = _PER_W // _NCHUNK


def _gather_body(idx_hbm, table_hbm, out_hbm, idx_v, rows_a, rows_b, sem_in, sem_a, sem_b):
    wid = lax.axis_index("s") * _NC + lax.axis_index("c")
    base = wid * _PER_W
    pltpu.async_copy(idx_hbm.at[pl.ds(base, _PER_W)], idx_v, sem_in).wait()

    bufs = (rows_a, rows_b)
    sems = (sem_a, sem_b)

    pltpu.async_copy(table_hbm.at[idx_v.at[pl.ds(0, _CH)]], rows_a, sem_a)
    for c in range(_NCHUNK):
        cur = bufs[c % 2]
        if c + 1 < _NCHUNK:
            pltpu.async_copy(
                table_hbm.at[idx_v.at[pl.ds((c + 1) * _CH, _CH)]],
                bufs[(c + 1) % 2], sems[(c + 1) % 2])
        pltpu.make_async_copy(table_hbm.at[idx_v.at[pl.ds(c * _CH, _CH)]],
                              cur, sems[c % 2]).wait()
        pltpu.sync_copy(cur, out_hbm.at[pl.ds(base + c * _CH, _CH)])


@jax.jit
def _sc_gather(idx_flat, table):
    mesh = plsc.VectorSubcoreMesh(core_axis_name="c", subcore_axis_name="s")
    k = pl.kernel(
        _gather_body,
        out_type=jax.ShapeDtypeStruct((_N, D), jnp.float32),
        mesh=mesh,
        scratch_types=[
            pltpu.VMEM((_PER_W,), jnp.int32),
            pltpu.VMEM((_CH, D), jnp.float32),
            pltpu.VMEM((_CH, D), jnp.float32),
            pltpu.SemaphoreType.DMA,
            pltpu.SemaphoreType.DMA,
            pltpu.SemaphoreType.DMA,
        ],
        compiler_params=pltpu.CompilerParams(use_tc_tiling_on_sc=False),
    )
    return k(idx_flat, table)


_CW = 6400


def _trans_body(t_ref, o_ref):
    x = t_ref[...]
    y = x.T
    y3 = y.reshape(_CW // 8, 8, 16)
    d16 = lax.iota(jnp.int32, 16)
    acc = None
    for a in range(8):
        onehot = (d16[:, None] + 16 * a == lax.iota(jnp.int32, 128)[None, :]
                  ).astype(jnp.float32)
        term = lax.dot_general(y3[:, a, :], onehot, (((1,), (0,)), ((), ())),
                               preferred_element_type=jnp.float32)
        acc = term if acc is None else acc + term
    o_ref[...] = acc


def _tc_transpose(tT):
    return pl.pallas_call(
        _trans_body,
        grid=((V + _CW - 1) // _CW,),
        in_specs=[pl.BlockSpec((16, _CW), lambda i: (0, i))],
        out_specs=pl.BlockSpec((_CW // 8, 128), lambda i: (i, 0)),
        out_shape=jax.ShapeDtypeStruct((V * D // 128, 128), jnp.float32),
        compiler_params=pltpu.CompilerParams(
            dimension_semantics=("arbitrary",),
        ),
    )(tT)


_BLK = 512


def _mlp_body(x_ref, scale_ref, shift_ref,
              w1_ref, b1_ref, w2_ref, b2_ref, w3_ref, b3_ref, o_ref):
    x4 = x_ref[...]
    xn = x4 * scale_ref[...][:, None, :] + shift_ref[...][:, None, :]
    h = b1_ref[...]
    for j in range(4):
        h = h + jnp.dot(xn[j], w1_ref[pl.ds(j * 128, 128), :],
                        preferred_element_type=jnp.float32)
    h = jnp.maximum(h, 0.0)
    h = jnp.dot(h, w2_ref[...], preferred_element_type=jnp.float32) + b2_ref[...]
    h = jnp.maximum(h, 0.0)
    h = jnp.dot(h, w3_ref[...], preferred_element_type=jnp.float32) + b3_ref[...]
    o_ref[...] = jnp.maximum(h, 0.0)


def _mlp(x3, scale4, shift4, W1p, b1, W2, b2, W3, b3):
    def vspec(n):
        return pl.BlockSpec((1, n), lambda i: (0, 0))

    return pl.pallas_call(
        _mlp_body,
        grid=(B // _BLK,),
        in_specs=[
            pl.BlockSpec((4, _BLK, 128), lambda i: (0, i, 0)),
            pl.BlockSpec((4, 128), lambda i: (0, 0)),
            pl.BlockSpec((4, 128), lambda i: (0, 0)),
            pl.BlockSpec((SDP, H1), lambda i: (0, 0)), vspec(H1),
            pl.BlockSpec((H1, H2), lambda i: (0, 0)), vspec(H2),
            pl.BlockSpec((H2, H3), lambda i: (0, 0)), vspec(H3),
        ],
        out_specs=pl.BlockSpec((_BLK, H3), lambda i: (i, 0)),
        out_shape=jax.ShapeDtypeStruct((B, H3), jnp.float32),
        compiler_params=pltpu.CompilerParams(
            dimension_semantics=("arbitrary",),
        ),
    )(x3, scale4, shift4, W1p, b1.reshape(1, H1), W2, b2.reshape(1, H2),
      W3, b3.reshape(1, H3))


def kernel(indices, table, gamma, beta, mean, var, W1, b1, W2, b2, W3, b3):
    fill = (jnp.arange(B, dtype=jnp.int32)[:, None] * (FP - F)
            + jnp.arange(FP - F, dtype=jnp.int32)[None, :]) % V
    idx_pad = jnp.concatenate([indices.astype(jnp.int32), fill], axis=1)
    idx_seq = idx_pad.reshape(B, 4, 8).transpose(1, 0, 2).reshape(-1)
    tbl_lin = _tc_transpose(table.T).reshape(V, D)
    rows = _sc_gather(idx_seq, tbl_lin)
    x3 = rows.reshape(4, B, 128)
    scale = gamma * lax.rsqrt(var + EPS)
    scale4 = jnp.pad(scale, (0, SDP - F * D)).reshape(4, 128)
    shift4 = jnp.pad(beta - mean * scale, (0, SDP - F * D)).reshape(4, 128)
    W1p = jnp.pad(W1, ((0, SDP - F * D), (0, 0)))
    return _mlp(x3, scale4, shift4, W1p, b1, W2, b2, W3, b3)


# speedup vs baseline: 1.0370x; 1.0370x over previous
import jax
import jax.numpy as jnp
from jax import lax
from jax.experimental import pallas as pl
from jax.experimental.pallas import tpu as pltpu, tpu_sc as plsc

B, F, V, D = 16384, 26, 1000000, 16
FP = 32
SDP = FP * D
H1, H2, H3 = 1024, 512, 256
EPS = 1e-5

_NC, _NS = 2, 16
_NW = _NC * _NS
_N = B * FP
_PER_W = _N // _NW
_NCHUNK = 8
_CH = _PER_W // _NCHUNK


def _gather_body(idx_hbm, table_hbm, out_hbm, idx_v, rows_a, rows_b, sem_in, sem_a, sem_b):
    wid = lax.axis_index("s") * _NC + lax.axis_index("c")
    base = wid * _PER_W
    pltpu.async_copy(idx_hbm.at[pl.ds(base, _PER_W)], idx_v, sem_in).wait()

    bufs = (rows_a, rows_b)
    sems = (sem_a, sem_b)

    pltpu.async_copy(table_hbm.at[idx_v.at[pl.ds(0, _CH)]], rows_a, sem_a)
    for c in range(_NCHUNK):
        cur = bufs[c % 2]
        if c + 1 < _NCHUNK:
            pltpu.async_copy(
                table_hbm.at[idx_v.at[pl.ds((c + 1) * _CH, _CH)]],
                bufs[(c + 1) % 2], sems[(c + 1) % 2])
        pltpu.make_async_copy(table_hbm.at[idx_v.at[pl.ds(c * _CH, _CH)]],
                              cur, sems[c % 2]).wait()
        pltpu.sync_copy(cur, out_hbm.at[pl.ds(base + c * _CH, _CH)])


@jax.jit
def _sc_gather(idx_flat, table):
    mesh = plsc.VectorSubcoreMesh(core_axis_name="c", subcore_axis_name="s")
    k = pl.kernel(
        _gather_body,
        out_type=jax.ShapeDtypeStruct((_N, D), jnp.float32),
        mesh=mesh,
        scratch_types=[
            pltpu.VMEM((_PER_W,), jnp.int32),
            pltpu.VMEM((_CH, D), jnp.float32),
            pltpu.VMEM((_CH, D), jnp.float32),
            pltpu.SemaphoreType.DMA,
            pltpu.SemaphoreType.DMA,
            pltpu.SemaphoreType.DMA,
        ],
        compiler_params=pltpu.CompilerParams(use_tc_tiling_on_sc=False),
    )
    return k(idx_flat, table)


_CW = 12800


def _trans_body(t_ref, o_ref):
    x = t_ref[...]
    y = x.T
    y3 = y.reshape(_CW // 8, 8, 16)
    d16 = lax.iota(jnp.int32, 16)
    acc = None
    for a in range(8):
        onehot = (d16[:, None] + 16 * a == lax.iota(jnp.int32, 128)[None, :]
                  ).astype(jnp.float32)
        term = lax.dot_general(y3[:, a, :], onehot, (((1,), (0,)), ((), ())),
                               preferred_element_type=jnp.float32)
        acc = term if acc is None else acc + term
    o_ref[...] = acc


def _tc_transpose(tT):
    return pl.pallas_call(
        _trans_body,
        grid=((V + _CW - 1) // _CW,),
        in_specs=[pl.BlockSpec((16, _CW), lambda i: (0, i))],
        out_specs=pl.BlockSpec((_CW // 8, 128), lambda i: (i, 0)),
        out_shape=jax.ShapeDtypeStruct((V * D // 128, 128), jnp.float32),
        compiler_params=pltpu.CompilerParams(
            dimension_semantics=("arbitrary",),
        ),
    )(tT)


_BLK = 512


def _mlp_body(x_ref, scale_ref, shift_ref,
              w1_ref, b1_ref, w2_ref, b2_ref, w3_ref, b3_ref, o_ref):
    x4 = x_ref[...]
    xn = x4 * scale_ref[...][:, None, :] + shift_ref[...][:, None, :]
    h = b1_ref[...]
    for j in range(4):
        h = h + jnp.dot(xn[j], w1_ref[pl.ds(j * 128, 128), :],
                        preferred_element_type=jnp.float32)
    h = jnp.maximum(h, 0.0)
    h = jnp.dot(h, w2_ref[...], preferred_element_type=jnp.float32) + b2_ref[...]
    h = jnp.maximum(h, 0.0)
    h = jnp.dot(h, w3_ref[...], preferred_element_type=jnp.float32) + b3_ref[...]
    o_ref[...] = jnp.maximum(h, 0.0)


def _mlp(x3, scale4, shift4, W1p, b1, W2, b2, W3, b3):
    def vspec(n):
        return pl.BlockSpec((1, n), lambda i: (0, 0))

    return pl.pallas_call(
        _mlp_body,
        grid=(B // _BLK,),
        in_specs=[
            pl.BlockSpec((4, _BLK, 128), lambda i: (0, i, 0)),
            pl.BlockSpec((4, 128), lambda i: (0, 0)),
            pl.BlockSpec((4, 128), lambda i: (0, 0)),
            pl.BlockSpec((SDP, H1), lambda i: (0, 0)), vspec(H1),
            pl.BlockSpec((H1, H2), lambda i: (0, 0)), vspec(H2),
            pl.BlockSpec((H2, H3), lambda i: (0, 0)), vspec(H3),
        ],
        out_specs=pl.BlockSpec((_BLK, H3), lambda i: (i, 0)),
        out_shape=jax.ShapeDtypeStruct((B, H3), jnp.float32),
        compiler_params=pltpu.CompilerParams(
            dimension_semantics=("arbitrary",),
        ),
    )(x3, scale4, shift4, W1p, b1.reshape(1, H1), W2, b2.reshape(1, H2),
      W3, b3.reshape(1, H3))


def kernel(indices, table, gamma, beta, mean, var, W1, b1, W2, b2, W3, b3):
    fill = (jnp.arange(B, dtype=jnp.int32)[:, None] * (FP - F)
            + jnp.arange(FP - F, dtype=jnp.int32)[None, :]) % V
    idx_pad = jnp.concatenate([indices.astype(jnp.int32), fill], axis=1)
    idx_seq = idx_pad.reshape(B, 4, 8).transpose(1, 0, 2).reshape(-1)
    tbl_lin = _tc_transpose(table.T).reshape(V, D)
    rows = _sc_gather(idx_seq, tbl_lin)
    x3 = rows.reshape(4, B, 128)
    scale = gamma * lax.rsqrt(var + EPS)
    scale4 = jnp.pad(scale, (0, SDP - F * D)).reshape(4, 128)
    shift4 = jnp.pad(beta - mean * scale, (0, SDP - F * D)).reshape(4, 128)
    W1p = jnp.pad(W1, ((0, SDP - F * D), (0, 0)))
    return _mlp(x3, scale4, shift4, W1p, b1, W2, b2, W3, b3)
